# baseline (speedup 1.0000x reference)
import functools

import jax
import jax.numpy as jnp
from jax import lax
from jax.experimental import pallas as pl
from jax.experimental.pallas import tpu as pltpu
from jax.experimental.pallas import tpu_sc as plsc

NC = 2
NS = 16
NW = NC * NS
LANES = 16

VB = 128
BCH = 2
SPLITS = (0, 96)
HPAD = 224


@functools.lru_cache(maxsize=None)
def _build_pad(V, D):
    n_full = V // VB
    tail = V % VB
    rem = n_full % NW
    per_w_max = -(-n_full // NW)
    n_pairs = -(-per_w_max // 2)

    mesh = plsc.VectorSubcoreMesh(core_axis_name="c", subcore_axis_name="s")

    def body(t_hbm, r_hbm, a0, a1, b0_, b1_, ia0, ia1, os0, os1):
        A = (a0, a1)
        B = (b0_, b1_)
        isem = (ia0, ia1)
        osem = (os0, os1)
        wid = lax.axis_index("s") * NC + lax.axis_index("c")
        cnt = (n_full // NW) + (wid < rem).astype(jnp.int32)

        def v0_of(i):
            c = jnp.minimum(i, cnt - 1) * NW + wid
            return c * VB

        def start_in(i, sl):
            pltpu.async_copy(t_hbm.at[pl.ds(v0_of(i), VB), :], A[sl], isem[sl])

        def wait_in(sl):
            pltpu.make_async_copy(
                t_hbm.at[pl.ds(0, VB), :], A[sl], isem[sl]).wait()

        def widen(sl, nrows):
            for r in range(nrows):
                for k in range(D // LANES):
                    B[sl][r, pl.ds(k * LANES, LANES)] = (
                        A[sl][r, pl.ds(k * LANES, LANES)])

        def start_out(i, sl):
            pltpu.async_copy(B[sl], r_hbm.at[pl.ds(v0_of(i), VB), :], osem[sl])

        def wait_out(sl):
            pltpu.make_async_copy(
                B[sl], r_hbm.at[pl.ds(0, VB), :], osem[sl]).wait()

        start_in(0, 0)
        start_in(1, 1)

        @pl.loop(0, n_pairs, step=1)
        def _steady(p):
            for b in range(2):
                i = p * 2 + b
                sl = b
                wait_in(sl)

                @pl.when(i >= 2)
                def _():
                    wait_out(sl)

                widen(sl, VB)
                start_out(i, sl)
                start_in(i + 2, sl)

        wait_in(0)
        wait_in(1)
        wait_out(0)
        wait_out(1)

        if tail:
            @pl.when(wid == NW - 1)
            def _tail():
                pltpu.async_copy(
                    t_hbm.at[pl.ds(V - tail, tail), :],
                    A[0].at[pl.ds(0, tail), :], isem[0])
                pltpu.make_async_copy(
                    t_hbm.at[pl.ds(0, tail), :],
                    A[0].at[pl.ds(0, tail), :], isem[0]).wait()
                for r in range(tail):
                    for k in range(D // LANES):
                        B[0][r, pl.ds(k * LANES, LANES)] = (
                            A[0][r, pl.ds(k * LANES, LANES)])
                pltpu.async_copy(
                    B[0].at[pl.ds(0, tail), :],
                    r_hbm.at[pl.ds(V - tail, tail), :], osem[0])
                pltpu.make_async_copy(
                    B[0].at[pl.ds(0, tail), :],
                    r_hbm.at[pl.ds(0, tail), :], osem[0]).wait()

    return pl.kernel(
        body,
        out_type=jax.ShapeDtypeStruct((V, 2 * D), jnp.float32),
        mesh=mesh,
        scratch_types=[
            pltpu.VMEM((VB, D), jnp.float32),
            pltpu.VMEM((VB, D), jnp.float32),
            pltpu.VMEM((VB, 2 * D), jnp.float32),
            pltpu.VMEM((VB, 2 * D), jnp.float32),
            pltpu.SemaphoreType.DMA,
            pltpu.SemaphoreType.DMA,
            pltpu.SemaphoreType.DMA,
            pltpu.SemaphoreType.DMA,
        ],
        compiler_params=pltpu.CompilerParams(use_tc_tiling_on_sc=True),
    )


HBLK = 8
CPB = HBLK // BCH


@functools.lru_cache(maxsize=None)
def _build_gather(B0, H, V, D):
    rows_per_w = B0 // NW
    n_blocks = rows_per_w // HBLK
    assert n_blocks % 2 == 0
    n_bpairs = n_blocks // 2
    n_chunks = n_blocks * CPB

    mesh = plsc.VectorSubcoreMesh(core_axis_name="c", subcore_axis_name="s")

    def body(xs_hbm, r_hbm, out_hbm,
             i0, i1, l0, l1, is0, is1, gs0, gs1, ss0, ss1):
        I = (i0, i1)
        L = (l0, l1)
        isem = (is0, is1)
        gsem = (gs0, gs1)
        ssem = (ss0, ss1)
        wid = lax.axis_index("s") * NC + lax.axis_index("c")
        b_base = wid * rows_per_w

        def start_idxblk(k, isl):
            off = pl.multiple_of((b_base + k * HBLK) * 2, 2 * HBLK)
            pltpu.async_copy(
                xs_hbm.at[pl.ds(off, 2 * HBLK), :], I[isl], isem[isl])

        def wait_idxblk(isl):
            pltpu.make_async_copy(
                xs_hbm.at[pl.ds(0, 2 * HBLK), :], I[isl], isem[isl]).wait()

        def start_gathers(isl, s, sl):
            for j in range(BCH):
                for t, o in enumerate(SPLITS):
                    pltpu.async_copy(
                        r_hbm.at[I[isl].at[(s * BCH + j) * 2 + t, :]],
                        L[sl].at[j, pl.ds(o, 128), :],
                        gsem[sl])

        def wait_gathers(sl):
            for j in range(BCH):
                for o in SPLITS:
                    pltpu.make_async_copy(
                        r_hbm.at[pl.ds(0, 128)], L[sl].at[j, pl.ds(o, 128), :],
                        gsem[sl]).wait()

        def start_store(g, sl):
            pltpu.async_copy(
                L[sl].at[:, pl.ds(0, H), :],
                out_hbm.at[pl.ds(b_base + g * BCH, BCH), :, :],
                ssem[sl])

        def wait_store(sl):
            pltpu.make_async_copy(
                L[sl].at[:, pl.ds(0, H), :],
                out_hbm.at[pl.ds(0, BCH), :, :], ssem[sl]).wait()

        start_idxblk(0, 0)
        start_idxblk(1, 1)
        wait_idxblk(0)
        start_gathers(0, 0, 0)

        @pl.loop(0, n_bpairs, step=1)
        def _steady(p):
            for kk in range(2):
                k = p * 2 + kk
                isl = kk
                for s in range(CPB):
                    g = k * CPB + s
                    sl = s % 2
                    nsl = 1 - sl

                    @pl.when(g >= 1)
                    def _():
                        wait_store(nsl)

                    if s < CPB - 1:
                        start_gathers(isl, s + 1, nsl)
                    else:
                        @pl.when(k < n_blocks - 1)
                        def _():
                            wait_idxblk(1 - isl)
                            start_gathers(1 - isl, 0, nsl)

                    wait_gathers(sl)
                    start_store(g, sl)
                    if s == CPB - 1:
                        @pl.when(k < n_blocks - 2)
                        def _():
                            start_idxblk(k + 2, isl)

        wait_store((CPB - 1) % 2)

    return pl.kernel(
        body,
        out_type=jax.ShapeDtypeStruct((B0, H, 2 * D), jnp.float32),
        mesh=mesh,
        scratch_types=[
            pltpu.VMEM((2 * HBLK, 128), jnp.int32),
            pltpu.VMEM((2 * HBLK, 128), jnp.int32),
            pltpu.VMEM((BCH, HPAD, 2 * D), jnp.float32),
            pltpu.VMEM((BCH, HPAD, 2 * D), jnp.float32),
            pltpu.SemaphoreType.DMA,
            pltpu.SemaphoreType.DMA,
            pltpu.SemaphoreType.DMA,
            pltpu.SemaphoreType.DMA,
            pltpu.SemaphoreType.DMA,
            pltpu.SemaphoreType.DMA,
        ],
        compiler_params=pltpu.CompilerParams(use_tc_tiling_on_sc=True),
    )


def kernel(x, table):
    B0, H = x.shape
    V, D = table.shape
    xp = jnp.pad(x, ((0, 0), (0, HPAD - H)))
    xs = jnp.concatenate(
        [xp[:, None, 0:128], xp[:, None, 96:224]], axis=1).reshape(2 * B0, 128)
    r = _build_pad(V, D)(table)
    out128 = _build_gather(B0, H, V, D)(xs, r)
    return out128[:, :, :D]


# speedup vs baseline: 1.0010x; 1.0010x over previous
import functools

import jax
import jax.numpy as jnp
from jax import lax
from jax.experimental import pallas as pl
from jax.experimental.pallas import tpu as pltpu
from jax.experimental.pallas import tpu_sc as plsc

NC = 2
NS = 16
NW = NC * NS
LANES = 16

VB = 128
BCH = 2
SPLITS = (0, 96)
HPAD = 224


@functools.lru_cache(maxsize=None)
def _build_pad(V, D):
    n_full = V // VB
    tail = V % VB
    rem = n_full % NW
    per_w_max = -(-n_full // NW)
    n_pairs = -(-per_w_max // 2)

    mesh = plsc.VectorSubcoreMesh(core_axis_name="c", subcore_axis_name="s")

    def body(t_hbm, r_hbm, a0, a1, b0_, b1_, ia0, ia1, os0, os1):
        A = (a0, a1)
        B = (b0_, b1_)
        isem = (ia0, ia1)
        osem = (os0, os1)
        wid = lax.axis_index("s") * NC + lax.axis_index("c")
        cnt = (n_full // NW) + (wid < rem).astype(jnp.int32)

        def v0_of(i):
            c = jnp.minimum(i, cnt - 1) * NW + wid
            return c * VB

        def start_in(i, sl):
            pltpu.async_copy(t_hbm.at[pl.ds(v0_of(i), VB), :], A[sl], isem[sl])

        def wait_in(sl):
            pltpu.make_async_copy(
                t_hbm.at[pl.ds(0, VB), :], A[sl], isem[sl]).wait()

        def widen(sl, nrows):
            for r in range(nrows):
                for k in range(D // LANES):
                    B[sl][r, pl.ds(k * LANES, LANES)] = (
                        A[sl][r, pl.ds(k * LANES, LANES)])

        def start_out(i, sl):
            pltpu.async_copy(B[sl], r_hbm.at[pl.ds(v0_of(i), VB), :], osem[sl])

        def wait_out(sl):
            pltpu.make_async_copy(
                B[sl], r_hbm.at[pl.ds(0, VB), :], osem[sl]).wait()

        start_in(0, 0)
        start_in(1, 1)

        @pl.loop(0, n_pairs, step=1)
        def _steady(p):
            for b in range(2):
                i = p * 2 + b
                sl = b
                wait_in(sl)

                @pl.when(i >= 2)
                def _():
                    wait_out(sl)

                widen(sl, VB)
                start_out(i, sl)
                start_in(i + 2, sl)

        wait_in(0)
        wait_in(1)
        wait_out(0)
        wait_out(1)

        if tail:
            @pl.when(wid == NW - 1)
            def _tail():
                pltpu.async_copy(
                    t_hbm.at[pl.ds(V - tail, tail), :],
                    A[0].at[pl.ds(0, tail), :], isem[0])
                pltpu.make_async_copy(
                    t_hbm.at[pl.ds(0, tail), :],
                    A[0].at[pl.ds(0, tail), :], isem[0]).wait()
                for r in range(tail):
                    for k in range(D // LANES):
                        B[0][r, pl.ds(k * LANES, LANES)] = (
                            A[0][r, pl.ds(k * LANES, LANES)])
                pltpu.async_copy(
                    B[0].at[pl.ds(0, tail), :],
                    r_hbm.at[pl.ds(V - tail, tail), :], osem[0])
                pltpu.make_async_copy(
                    B[0].at[pl.ds(0, tail), :],
                    r_hbm.at[pl.ds(0, tail), :], osem[0]).wait()

    return pl.kernel(
        body,
        out_type=jax.ShapeDtypeStruct((V, 2 * D), jnp.float32),
        mesh=mesh,
        scratch_types=[
            pltpu.VMEM((VB, D), jnp.float32),
            pltpu.VMEM((VB, D), jnp.float32),
            pltpu.VMEM((VB, 2 * D), jnp.float32),
            pltpu.VMEM((VB, 2 * D), jnp.float32),
            pltpu.SemaphoreType.DMA,
            pltpu.SemaphoreType.DMA,
            pltpu.SemaphoreType.DMA,
            pltpu.SemaphoreType.DMA,
        ],
        compiler_params=pltpu.CompilerParams(use_tc_tiling_on_sc=True),
    )


HBLK = 8
CPB = HBLK // BCH


@functools.lru_cache(maxsize=None)
def _build_gather(B0, H, V, D):
    rows_per_w = B0 // NW
    n_blocks = rows_per_w // HBLK
    assert n_blocks % 2 == 0
    n_bpairs = n_blocks // 2
    n_chunks = n_blocks * CPB

    mesh = plsc.VectorSubcoreMesh(core_axis_name="c", subcore_axis_name="s")

    def body(xs_hbm, r_hbm, out_hbm,
             i0, i1, l0, l1, is0, is1, gs0, gs1, ss0, ss1):
        I = (i0, i1)
        L = (l0, l1)
        isem = (is0, is1)
        gsem = (gs0, gs1)
        ssem = (ss0, ss1)
        wid = lax.axis_index("s") * NC + lax.axis_index("c")
        b_base = wid * rows_per_w

        def start_idxblk(k, isl):
            off = pl.multiple_of((b_base + k * HBLK) * 2, 2 * HBLK)
            pltpu.async_copy(
                xs_hbm.at[pl.ds(off, 2 * HBLK), :], I[isl], isem[isl])

        def wait_idxblk(isl):
            pltpu.make_async_copy(
                xs_hbm.at[pl.ds(0, 2 * HBLK), :], I[isl], isem[isl]).wait()

        def start_gathers(isl, s, sl):
            for j in range(BCH):
                for t, o in enumerate(SPLITS):
                    pltpu.async_copy(
                        r_hbm.at[I[isl].at[(s * BCH + j) * 2 + t, :]],
                        L[sl].at[j, pl.ds(o, 128), :],
                        gsem[sl])

        def wait_gathers(sl):
            for j in range(BCH):
                for o in SPLITS:
                    pltpu.make_async_copy(
                        r_hbm.at[pl.ds(0, 128)], L[sl].at[j, pl.ds(o, 128), :],
                        gsem[sl]).wait()

        def start_store(g, sl):
            pltpu.async_copy(
                L[sl].at[:, pl.ds(0, H), :],
                out_hbm.at[pl.ds(b_base + g * BCH, BCH), :, :],
                ssem[sl])

        def wait_store(sl):
            pltpu.make_async_copy(
                L[sl].at[:, pl.ds(0, H), :],
                out_hbm.at[pl.ds(0, BCH), :, :], ssem[sl]).wait()

        start_idxblk(0, 0)
        start_idxblk(1, 1)
        wait_idxblk(0)
        start_gathers(0, 0, 0)

        @pl.loop(0, n_bpairs, step=1)
        def _steady(p):
            for kk in range(2):
                k = p * 2 + kk
                isl = kk
                for s in range(CPB):
                    g = k * CPB + s
                    sl = s % 2
                    nsl = 1 - sl

                    @pl.when(g >= 1)
                    def _():
                        wait_store(nsl)

                    if s < CPB - 1:
                        start_gathers(isl, s + 1, nsl)
                    else:
                        @pl.when(k < n_blocks - 1)
                        def _():
                            wait_idxblk(1 - isl)
                            start_gathers(1 - isl, 0, nsl)

                    wait_gathers(sl)
                    start_store(g, sl)
                    if s == CPB - 1:
                        @pl.when(k < n_blocks - 2)
                        def _():
                            start_idxblk(k + 2, isl)

        wait_store((CPB - 1) % 2)

    return pl.kernel(
        body,
        out_type=jax.ShapeDtypeStruct((B0, H, 2 * D), jnp.float32),
        mesh=mesh,
        scratch_types=[
            pltpu.VMEM((2 * HBLK, 128), jnp.int32),
            pltpu.VMEM((2 * HBLK, 128), jnp.int32),
            pltpu.VMEM((BCH, HPAD, 2 * D), jnp.float32),
            pltpu.VMEM((BCH, HPAD, 2 * D), jnp.float32),
            pltpu.SemaphoreType.DMA,
            pltpu.SemaphoreType.DMA,
            pltpu.SemaphoreType.DMA,
            pltpu.SemaphoreType.DMA,
            pltpu.SemaphoreType.DMA,
            pltpu.SemaphoreType.DMA,
        ],
        compiler_params=pltpu.CompilerParams(use_tc_tiling_on_sc=False),
    )


def kernel(x, table):
    B0, H = x.shape
    V, D = table.shape
    xp = jnp.pad(x, ((0, 0), (0, HPAD - H)))
    xs = jnp.concatenate(
        [xp[:, None, 0:128], xp[:, None, 96:224]], axis=1).reshape(2 * B0, 128)
    r = _build_pad(V, D)(table)
    out128 = _build_gather(B0, H, V, D)(xs, r)
    return out128[:, :, :D]


# speedup vs baseline: 6.4457x; 6.4394x over previous
import functools

import jax
import jax.numpy as jnp
from jax import lax
from jax.experimental import pallas as pl
from jax.experimental.pallas import tpu as pltpu
from jax.experimental.pallas import tpu_sc as plsc

NC = 2
NS = 16
NW = NC * NS

BCH = 4
SPLITS = ((0, 96), (96, 104))


@functools.lru_cache(maxsize=None)
def _build(B0, H, V, D):
    assert B0 % (NW * BCH) == 0
    b_per_w = B0 // NW
    n_iter = b_per_w // BCH
    assert n_iter >= 4 and (n_iter - 2) % 2 == 0

    mesh = plsc.VectorSubcoreMesh(core_axis_name="c", subcore_axis_name="s")

    def body(x_hbm, table_hbm, out_hbm,
             i0, i1, r0, r1, is0, is1, gs0, gs1, ss0, ss1):
        I = (i0, i1)
        R = (r0, r1)
        isem = (is0, is1)
        gsem = (gs0, gs1)
        ssem = (ss0, ss1)
        wid = lax.axis_index("s") * NC + lax.axis_index("c")
        b_base = wid * b_per_w

        def start_idx(g, sl):
            pltpu.async_copy(
                x_hbm.at[pl.ds(b_base + g * BCH, BCH), :], I[sl], isem[sl])

        def wait_idx(sl):
            pltpu.make_async_copy(
                x_hbm.at[pl.ds(0, BCH), :], I[sl], isem[sl]).wait()

        def start_gathers(sl):
            for j in range(BCH):
                for (o, n) in SPLITS:
                    pltpu.async_copy(
                        table_hbm.at[I[sl].at[j, pl.ds(o, n)]],
                        R[sl].at[j, pl.ds(o, n)],
                        gsem[sl])

        def wait_gathers(sl):
            pltpu.make_async_copy(
                out_hbm.at[pl.ds(0, BCH), :, :], R[sl], gsem[sl]).wait()

        def start_store(g, sl):
            pltpu.async_copy(
                R[sl], out_hbm.at[pl.ds(b_base + g * BCH, BCH), :, :],
                ssem[sl])

        def wait_store(sl):
            pltpu.make_async_copy(
                R[sl], out_hbm.at[pl.ds(0, BCH), :, :], ssem[sl]).wait()

        start_idx(0, 0)
        start_idx(1, 1)
        wait_idx(0)
        start_gathers(0)
        wait_idx(1)
        start_gathers(1)
        wait_gathers(0)
        start_store(0, 0)
        start_idx(2, 0)

        @pl.loop(1, n_iter - 1, step=2)
        def _steady(base):
            for b in range(2):
                g = base + b
                sl = (1 + b) % 2
                nsl = 1 - sl
                wait_store(nsl)
                wait_idx(nsl)
                start_gathers(nsl)
                wait_gathers(sl)
                start_store(g, sl)
                gnext = jnp.minimum(g + 2, n_iter - 1)
                start_idx(gnext, sl)

        fb = (n_iter - 1) % 2
        nfb = 1 - fb
        wait_store(nfb)
        wait_idx(nfb)
        wait_gathers(fb)
        start_store(n_iter - 1, fb)
        wait_store(fb)

    return pl.kernel(
        body,
        out_type=jax.ShapeDtypeStruct((B0, H, D), jnp.float32),
        mesh=mesh,
        scratch_types=[
            pltpu.VMEM((BCH, H), jnp.int32),
            pltpu.VMEM((BCH, H), jnp.int32),
            pltpu.VMEM((BCH, H, D), jnp.float32),
            pltpu.VMEM((BCH, H, D), jnp.float32),
            pltpu.SemaphoreType.DMA,
            pltpu.SemaphoreType.DMA,
            pltpu.SemaphoreType.DMA,
            pltpu.SemaphoreType.DMA,
            pltpu.SemaphoreType.DMA,
            pltpu.SemaphoreType.DMA,
        ],
        compiler_params=pltpu.CompilerParams(use_tc_tiling_on_sc=False),
    )


def kernel(x, table):
    B0, H = x.shape
    V, D = table.shape
    return _build(B0, H, V, D)(x, table)
